# baseline (speedup 1.0000x reference)
import functools
import jax
import jax.numpy as jnp
from jax import lax
from jax.experimental import pallas as pl
from jax.experimental.pallas import tpu as pltpu
from jax.experimental.pallas import tpu_sc as plsc

_COLS = (0, 1, 2, 6, 7)
_EMB = 128
_NSEL = 8
_NFEAT = 5
_NC = 2
_NS = 16
_NW = _NC * _NS
_CHUNK = 80
_LANES = 16


def _sc_call(dat_flat, ctab, n_tok):
    tpw = n_tok // _NW
    nchunk = tpw // _CHUNK
    rows = _CHUNK * _NFEAT
    mesh = plsc.VectorSubcoreMesh(core_axis_name="c", subcore_axis_name="s")

    @functools.partial(
        pl.kernel,
        mesh=mesh,
        out_type=jax.ShapeDtypeStruct((n_tok, _NFEAT * _EMB), jnp.float32),
        scratch_types=[
            pltpu.VMEM((rows,), jnp.int32),
            pltpu.VMEM((rows,), jnp.int32),
            pltpu.VMEM((rows,), jnp.int32),
            pltpu.VMEM((rows,), jnp.int32),
            pltpu.VMEM((rows, _EMB), jnp.float32),
            pltpu.VMEM((rows, _EMB), jnp.float32),
            pltpu.VMEM_SHARED((_NFEAT * _NSEL, _EMB), jnp.float32),
            pltpu.SemaphoreType.DMA,
            pltpu.SemaphoreType.DMA,
            pltpu.SemaphoreType.DMA,
        ],
    )
    def k(dat_hbm, ctab_hbm, out_hbm, dat0, dat1, idx0, idx1, rows0, rows1,
          ctab_s, gsem, ssem0, ssem1):
        cid = lax.axis_index("c")
        sid = lax.axis_index("s")
        wid = sid * _NC + cid
        base_tok = wid * tpw
        bufs = ((dat0, idx0, rows0, ssem0), (dat1, idx1, rows1, ssem1))

        @pl.when(sid == 0)
        def _():
            pltpu.sync_copy(ctab_hbm, ctab_s)

        plsc.subcore_barrier()

        def do_chunk(ci, b):
            dat_v, idx_v, rows_v, ssem = bufs[b]
            t0 = base_tok + ci * _CHUNK
            pltpu.sync_copy(dat_hbm.at[pl.ds(t0 * _NFEAT, rows)], dat_v)
            for f in range(_NFEAT):
                for g in range(_CHUNK // _LANES):
                    o = f * _CHUNK + g * _LANES
                    idx_v[pl.ds(o, _LANES)] = dat_v[pl.ds(o, _LANES)] + f * _NSEL
            gathers = [
                pltpu.async_copy(
                    ctab_s.at[idx_v.at[pl.ds(f * _CHUNK, _CHUNK)]],
                    rows_v.at[pl.ds(f * _CHUNK, _CHUNK)],
                    gsem,
                )
                for f in range(_NFEAT)
            ]
            for h in gathers:
                h.wait()
            half = _CHUNK // 2
            for f in range(_NFEAT):
                for h in range(2):
                    pltpu.async_copy(
                        rows_v.at[pl.ds(f * _CHUNK + h * half, half)],
                        out_hbm.at[pl.ds(t0 + h * half, half), pl.ds(f * _EMB, _EMB)],
                        ssem,
                    )

        def drain(b):
            _, _, rows_v, ssem = bufs[b]
            for _f in range(_NFEAT * 2):
                pltpu.make_async_copy(
                    rows_v.at[pl.ds(0, _CHUNK // 2)],
                    out_hbm.at[pl.ds(0, _CHUNK // 2), pl.ds(0, _EMB)],
                    ssem,
                ).wait()

        def pair(p, _):
            for b in range(2):
                @pl.when(p > 0)
                def _():
                    drain(b)

                do_chunk(p * 2 + b, b)
            return ()

        lax.fori_loop(0, nchunk // 2, pair, ())
        for b in range(2):
            drain(b)

    return k(dat_flat, ctab)


def kernel(data, user_table, poi_table, category_table, dayofweek_table, hourofday_table):
    n_tok = data.shape[0] * data.shape[1]
    cols = jnp.array(_COLS, jnp.int32)
    dat5 = data.reshape(n_tok, 8).astype(jnp.int32)[:, cols].T
    dat_flat = (
        dat5.reshape(_NFEAT, n_tok // _CHUNK, _CHUNK)
        .transpose(1, 0, 2)
        .reshape(-1)
    )
    tabs = [user_table, poi_table, category_table, dayofweek_table, hourofday_table]
    ctab = jnp.concatenate([t[:_NSEL].at[0].set(0.0) for t in tabs], axis=0)
    out = _sc_call(dat_flat, ctab, n_tok)
    return out.reshape(data.shape[0], data.shape[1], _NFEAT * _EMB)


# speedup vs baseline: 1.1107x; 1.1107x over previous
import functools
import jax
import jax.numpy as jnp
from jax import lax
from jax.experimental import pallas as pl
from jax.experimental.pallas import tpu as pltpu
from jax.experimental.pallas import tpu_sc as plsc

_COLS = (0, 1, 2, 6, 7)
_EMB = 128
_NSEL = 8
_NFEAT = 5
_NC = 2
_NS = 16
_NW = _NC * _NS
_CHUNK = 80
_LANES = 16


def _sc_call(dat_flat, ctab, n_tok):
    tpw = n_tok // _NW
    nchunk = tpw // _CHUNK
    rows = _CHUNK * _NFEAT
    mesh = plsc.VectorSubcoreMesh(core_axis_name="c", subcore_axis_name="s")

    @functools.partial(
        pl.kernel,
        mesh=mesh,
        out_type=jax.ShapeDtypeStruct((n_tok, _NFEAT * _EMB), jnp.float32),
        scratch_types=[
            pltpu.VMEM((rows,), jnp.int32),
            pltpu.VMEM((rows,), jnp.int32),
            pltpu.VMEM((rows,), jnp.int32),
            pltpu.VMEM((rows,), jnp.int32),
            pltpu.VMEM((rows, _EMB), jnp.float32),
            pltpu.VMEM((rows, _EMB), jnp.float32),
            pltpu.VMEM_SHARED((_NFEAT * _NSEL, _EMB), jnp.float32),
            pltpu.SemaphoreType.DMA,
            pltpu.SemaphoreType.DMA,
            pltpu.SemaphoreType.DMA,
            pltpu.SemaphoreType.DMA,
            pltpu.SemaphoreType.DMA,
        ],
    )
    def k(dat_hbm, ctab_hbm, out_hbm, dat0, dat1, idx0, idx1, rows0, rows1,
          ctab_s, gsem, ssem0, ssem1, dsem0, dsem1):
        cid = lax.axis_index("c")
        sid = lax.axis_index("s")
        wid = sid * _NC + cid
        base_tok = wid * tpw
        bufs = ((dat0, idx0, rows0, ssem0, dsem0), (dat1, idx1, rows1, ssem1, dsem1))

        def prefetch(ci, b):
            dat_v, _, _, _, dsem = bufs[b]
            @pl.when(ci < nchunk)
            def _():
                t0 = base_tok + ci * _CHUNK
                pltpu.async_copy(dat_hbm.at[pl.ds(t0 * _NFEAT, rows)], dat_v, dsem)

        @pl.when(sid == 0)
        def _():
            pltpu.sync_copy(ctab_hbm, ctab_s)

        plsc.subcore_barrier()

        def do_chunk(ci, b):
            dat_v, idx_v, rows_v, ssem, dsem = bufs[b]
            t0 = base_tok + ci * _CHUNK
            pltpu.make_async_copy(
                dat_hbm.at[pl.ds(0, rows)], dat_v, dsem
            ).wait()
            for f in range(_NFEAT):
                for g in range(_CHUNK // _LANES):
                    o = f * _CHUNK + g * _LANES
                    idx_v[pl.ds(o, _LANES)] = dat_v[pl.ds(o, _LANES)] + f * _NSEL
            prefetch(ci + 2, b)
            gathers = [
                pltpu.async_copy(
                    ctab_s.at[idx_v.at[pl.ds(f * _CHUNK, _CHUNK)]],
                    rows_v.at[pl.ds(f * _CHUNK, _CHUNK)],
                    gsem,
                )
                for f in range(_NFEAT)
            ]
            for h in gathers:
                h.wait()
            for f in range(_NFEAT):
                pltpu.async_copy(
                    rows_v.at[pl.ds(f * _CHUNK, _CHUNK)],
                    out_hbm.at[pl.ds(t0, _CHUNK), pl.ds(f * _EMB, _EMB)],
                    ssem,
                )

        def drain(b):
            _, _, rows_v, ssem, _ = bufs[b]
            for _f in range(_NFEAT):
                pltpu.make_async_copy(
                    rows_v.at[pl.ds(0, _CHUNK)],
                    out_hbm.at[pl.ds(0, _CHUNK), pl.ds(0, _EMB)],
                    ssem,
                ).wait()

        def pair(p, _):
            for b in range(2):
                @pl.when(p > 0)
                def _():
                    drain(b)

                do_chunk(p * 2 + b, b)
            return ()

        prefetch(0, 0)
        prefetch(1, 1)
        lax.fori_loop(0, nchunk // 2, pair, ())
        for b in range(2):
            drain(b)

    return k(dat_flat, ctab)


def kernel(data, user_table, poi_table, category_table, dayofweek_table, hourofday_table):
    n_tok = data.shape[0] * data.shape[1]
    cols = jnp.array(_COLS, jnp.int32)
    dat5 = data.reshape(n_tok, 8).astype(jnp.int32)[:, cols].T
    dat_flat = (
        dat5.reshape(_NFEAT, n_tok // _CHUNK, _CHUNK)
        .transpose(1, 0, 2)
        .reshape(-1)
    )
    tabs = [user_table, poi_table, category_table, dayofweek_table, hourofday_table]
    ctab = jnp.concatenate([t[:_NSEL].at[0].set(0.0) for t in tabs], axis=0)
    out = _sc_call(dat_flat, ctab, n_tok)
    return out.reshape(data.shape[0], data.shape[1], _NFEAT * _EMB)


# speedup vs baseline: 1.1346x; 1.0215x over previous
import functools
import jax
import jax.numpy as jnp
from jax import lax
from jax.experimental import pallas as pl
from jax.experimental.pallas import tpu as pltpu
from jax.experimental.pallas import tpu_sc as plsc

_COLS = (0, 1, 2, 6, 7)
_EMB = 128
_NSEL = 8
_NFEAT = 5
_NC = 2
_NS = 16
_NW = _NC * _NS
_CHUNK = 80
_LANES = 16


def _sc_call(dat_flat, ctab, n_tok):
    tpw = n_tok // _NW
    nchunk = tpw // _CHUNK
    rows = _CHUNK * _NFEAT
    mesh = plsc.VectorSubcoreMesh(core_axis_name="c", subcore_axis_name="s")

    @functools.partial(
        pl.kernel,
        mesh=mesh,
        out_type=jax.ShapeDtypeStruct((n_tok, _NFEAT * _EMB), jnp.float32),
        scratch_types=[
            pltpu.VMEM((rows,), jnp.int32),
            pltpu.VMEM((rows,), jnp.int32),
            pltpu.VMEM((rows,), jnp.int32),
            pltpu.VMEM((rows,), jnp.int32),
            pltpu.VMEM((rows, _EMB), jnp.float32),
            pltpu.VMEM((rows, _EMB), jnp.float32),
            pltpu.VMEM_SHARED((_NFEAT * _NSEL, _EMB), jnp.float32),
            pltpu.SemaphoreType.DMA,
            pltpu.SemaphoreType.DMA,
            pltpu.SemaphoreType.DMA,
            pltpu.SemaphoreType.DMA,
            pltpu.SemaphoreType.DMA,
        ],
    )
    def k(dat_hbm, ctab_hbm, out_hbm, dat0, dat1, idx0, idx1, rows0, rows1,
          ctab_s, gsem, ssem0, ssem1, dsem0, dsem1):
        cid = lax.axis_index("c")
        sid = lax.axis_index("s")
        wid = sid * _NC + cid
        base_tok = wid * tpw
        bufs = ((dat0, idx0, rows0, ssem0, dsem0), (dat1, idx1, rows1, ssem1, dsem1))

        def prefetch(ci, b):
            dat_v, _, _, _, dsem = bufs[b]
            @pl.when(ci < nchunk)
            def _():
                t0 = base_tok + ci * _CHUNK
                pltpu.async_copy(dat_hbm.at[pl.ds(t0 * _NFEAT, rows)], dat_v, dsem)

        @pl.when(sid == 0)
        def _():
            pltpu.sync_copy(ctab_hbm, ctab_s)

        plsc.subcore_barrier()

        def do_chunk(ci, b):
            dat_v, idx_v, rows_v, ssem, dsem = bufs[b]
            t0 = base_tok + ci * _CHUNK
            pltpu.make_async_copy(
                dat_hbm.at[pl.ds(0, rows)], dat_v, dsem
            ).wait()
            for f in range(_NFEAT):
                for g in range(_CHUNK // _LANES):
                    o = f * _CHUNK + g * _LANES
                    idx_v[pl.ds(o, _LANES)] = dat_v[pl.ds(o, _LANES)] + f * _NSEL
            prefetch(ci + 2, b)
            gathers = [
                pltpu.async_copy(
                    ctab_s.at[idx_v.at[pl.ds(f * _CHUNK, _CHUNK)]],
                    rows_v.at[pl.ds(f * _CHUNK, _CHUNK)],
                    gsem,
                )
                for f in range(_NFEAT)
            ]
            for f in range(_NFEAT):
                gathers[f].wait()
                pltpu.async_copy(
                    rows_v.at[pl.ds(f * _CHUNK, _CHUNK)],
                    out_hbm.at[pl.ds(t0, _CHUNK), pl.ds(f * _EMB, _EMB)],
                    ssem,
                )

        def drain(b):
            _, _, rows_v, ssem, _ = bufs[b]
            for _f in range(_NFEAT):
                pltpu.make_async_copy(
                    rows_v.at[pl.ds(0, _CHUNK)],
                    out_hbm.at[pl.ds(0, _CHUNK), pl.ds(0, _EMB)],
                    ssem,
                ).wait()

        def pair(p, _):
            for b in range(2):
                @pl.when(p > 0)
                def _():
                    drain(b)

                do_chunk(p * 2 + b, b)
            return ()

        prefetch(0, 0)
        prefetch(1, 1)
        lax.fori_loop(0, nchunk // 2, pair, ())
        for b in range(2):
            drain(b)

    return k(dat_flat, ctab)


def kernel(data, user_table, poi_table, category_table, dayofweek_table, hourofday_table):
    n_tok = data.shape[0] * data.shape[1]
    cols = jnp.array(_COLS, jnp.int32)
    dat5 = data.reshape(n_tok, 8).astype(jnp.int32)[:, cols].T
    dat_flat = (
        dat5.reshape(_NFEAT, n_tok // _CHUNK, _CHUNK)
        .transpose(1, 0, 2)
        .reshape(-1)
    )
    tabs = [user_table, poi_table, category_table, dayofweek_table, hourofday_table]
    ctab = jnp.concatenate([t[:_NSEL].at[0].set(0.0) for t in tabs], axis=0)
    out = _sc_call(dat_flat, ctab, n_tok)
    return out.reshape(data.shape[0], data.shape[1], _NFEAT * _EMB)
